# Initial kernel scaffold; baseline (speedup 1.0000x reference)
import jax
import jax.numpy as jnp
from jax.experimental import pallas as pl


def kernel(x, edge_index, edges_pos, edges_neg, W):
    raise NotImplementedError("write your pallas kernel here")


# speedup vs baseline: 1.3671x; 1.3671x over previous
import functools

import jax
import jax.numpy as jnp
from jax import lax
from jax.experimental import pallas as pl
from jax.experimental.pallas import tpu as pltpu
from jax.experimental.pallas import tpu_sc as plsc

N_NODES = 10000
D = 128
N_EDGES = 320000

NC = 2
NS = 16
L = 16

N_HALF = N_NODES // NC
ZP = 5120
TRASH = 5100
ZROWS_PER_TILE = ZP // NS

EI_ROWS = 2560
EI_ROWS_PER_TILE = EI_ROWS // NS
K2_CHUNK_ROWS = 4
K2_CHUNKS = EI_ROWS_PER_TILE // K2_CHUNK_ROWS

SC_ROWS = 5120
SC_ROWS_PER_W = SC_ROWS // (NC * NS)
K3_CHUNK_ROWS = 1
K3_CHUNKS = SC_ROWS_PER_W // K3_CHUNK_ROWS
POS_VALID_ROWS = N_EDGES // 128


def _mesh():
  return plsc.VectorSubcoreMesh(
      core_axis_name="c", subcore_axis_name="s", num_cores=NC,
      num_subcores=NS)


def _mm_body(x_ref, w_ref, o_ref):
  o_ref[...] = lax.dot_general(
      x_ref[...], w_ref[...], (((1,), (0,)), ((), ())),
      preferred_element_type=jnp.float32,
      precision=lax.Precision.HIGHEST)


def _matmul(x, w):
  m_blk = 1000
  return pl.pallas_call(
      _mm_body,
      grid=(N_NODES // m_blk,),
      in_specs=[
          pl.BlockSpec((m_blk, D), lambda i: (i, 0)),
          pl.BlockSpec((D, D), lambda i: (0, 0)),
      ],
      out_specs=pl.BlockSpec((m_blk, D), lambda i: (i, 0)),
      out_shape=jax.ShapeDtypeStruct((N_NODES, D), jnp.float32),
  )(x, w)


def _segsum_body(h_ref, ei_ref, zinit_ref, z_out, src_v, dst_v, loc_v,
                 rows_v, z_sh, sem):
  c = lax.axis_index("c")
  s = lax.axis_index("s")

  pltpu.sync_copy(zinit_ref, z_sh.at[pl.ds(s * ZROWS_PER_TILE,
                                           ZROWS_PER_TILE)])
  plsc.subcore_barrier()

  base = c * N_HALF

  def chunk(k, carry):
    row0 = s * EI_ROWS_PER_TILE + k * K2_CHUNK_ROWS
    pltpu.sync_copy(ei_ref.at[0, pl.ds(row0, K2_CHUNK_ROWS)], src_v)
    pltpu.sync_copy(ei_ref.at[1, pl.ds(row0, K2_CHUNK_ROWS)], dst_v)
    descs = [
        pltpu.async_copy(h_ref.at[src_v.at[j]],
                         rows_v.at[pl.ds(j * 128, 128)], sem)
        for j in range(K2_CHUNK_ROWS)
    ]
    for j in range(K2_CHUNK_ROWS):
      for i in range(128 // L):
        d = dst_v[j, pl.ds(i * L, L)]
        dloc = d - base
        inb = (dloc >= 0) & (dloc < N_HALF)
        loc_v[j, pl.ds(i * L, L)] = jnp.where(inb, dloc, TRASH)
    for dsc in descs:
      dsc.wait()
    for j in range(K2_CHUNK_ROWS):
      pltpu.sync_copy(rows_v.at[pl.ds(j * 128, 128)],
                      z_sh.at[loc_v.at[j]], add=True)
    return carry

  lax.fori_loop(0, K2_CHUNKS, chunk, 0)
  plsc.subcore_barrier()

  pltpu.sync_copy(
      z_sh.at[pl.ds(s * ZROWS_PER_TILE, ZROWS_PER_TILE)],
      z_out.at[pl.ds(c * ZP + s * ZROWS_PER_TILE, ZROWS_PER_TILE)])


def _segsum(h, ei3, zinit):
  return pl.kernel(
      _segsum_body,
      out_type=jax.ShapeDtypeStruct((NC * ZP, D), jnp.float32),
      mesh=_mesh(),
      scratch_types=[
          pltpu.VMEM((K2_CHUNK_ROWS, 128), jnp.int32),
          pltpu.VMEM((K2_CHUNK_ROWS, 128), jnp.int32),
          pltpu.VMEM((K2_CHUNK_ROWS, 128), jnp.int32),
          pltpu.VMEM((K2_CHUNK_ROWS * 128, D), jnp.float32),
          pltpu.VMEM_SHARED((ZP, D), jnp.float32),
          pltpu.SemaphoreType.DMA,
      ],
  )(h, ei3, zinit)


def _scores_body(z_ref, eidx_ref, p_out, a_v, b_v, za_v, zb_v, s_v, sem):
  c = lax.axis_index("c")
  s = lax.axis_index("s")
  w = s * NC + c

  def chunk(k, carry):
    row0 = w * SC_ROWS_PER_W + k
    pltpu.sync_copy(eidx_ref.at[0, pl.ds(row0, 1)], a_v)
    pltpu.sync_copy(eidx_ref.at[1, pl.ds(row0, 1)], b_v)
    for i in range(128 // L):
      a = a_v[0, pl.ds(i * L, L)]
      a_v[0, pl.ds(i * L, L)] = jnp.where(a >= N_HALF, a + (ZP - N_HALF), a)
      b = b_v[0, pl.ds(i * L, L)]
      b_v[0, pl.ds(i * L, L)] = jnp.where(b >= N_HALF, b + (ZP - N_HALF), b)
    descs = [
        pltpu.async_copy(z_ref.at[a_v.at[0]], za_v, sem),
        pltpu.async_copy(z_ref.at[b_v.at[0]], zb_v, sem),
    ]
    for dsc in descs:
      dsc.wait()
    for e in range(128):
      acc = za_v[e, pl.ds(0, L)] * zb_v[e, pl.ds(0, L)]
      for q in range(1, D // L):
        acc = acc + za_v[e, pl.ds(q * L, L)] * zb_v[e, pl.ds(q * L, L)]
      s_v[pl.ds(e * L, L)] = acc
    pltpu.sync_copy(s_v, p_out.at[pl.ds(row0 * 128 * L, 128 * L)])
    return carry

  lax.fori_loop(0, K3_CHUNKS, chunk, 0)


def _scores(z, eidx3):
  return pl.kernel(
      _scores_body,
      out_type=jax.ShapeDtypeStruct((SC_ROWS * 128 * L,), jnp.float32),
      mesh=_mesh(),
      scratch_types=[
          pltpu.VMEM((K3_CHUNK_ROWS, 128), jnp.int32),
          pltpu.VMEM((K3_CHUNK_ROWS, 128), jnp.int32),
          pltpu.VMEM((K3_CHUNK_ROWS * 128, D), jnp.float32),
          pltpu.VMEM((K3_CHUNK_ROWS * 128, D), jnp.float32),
          pltpu.VMEM((K3_CHUNK_ROWS * 128 * L,), jnp.float32),
          pltpu.SemaphoreType.DMA,
      ],
  )(z, eidx3)


E_HALF_PAD = SC_ROWS // 2 * 128


def _bce_body(p_ref, m_ref, o_ref):
  i = pl.program_id(0)
  nrows = p_ref.shape[0]
  scores = lax.dot_general(
      p_ref[...], m_ref[...], (((1,), (0,)), ((), ())),
      preferred_element_type=jnp.float32,
      precision=lax.Precision.HIGHEST)
  r = lax.broadcasted_iota(jnp.int32, scores.shape, 0) + i * nrows
  g = lax.broadcasted_iota(jnp.int32, scores.shape, 1)
  eg = r * 8 + g
  is_pos = eg < E_HALF_PAD
  valid = (eg < N_EDGES) | ((eg >= E_HALF_PAD) &
                            (eg < E_HALF_PAD + N_EDGES))
  t = jnp.where(is_pos, 1.0, 0.0)
  p = scores
  term = jnp.maximum(p, 0.0) - p * t + jnp.log1p(jnp.exp(-jnp.abs(p)))
  term = jnp.where(valid, term, 0.0)
  bsum = jnp.sum(term)
  prev = jnp.where(i == 0, 0.0, o_ref[0, 0])
  tot = prev + bsum
  o_ref[0, 0] = jnp.where(i == pl.num_programs(0) - 1,
                          tot / (2.0 * N_EDGES), tot)


def _bce(partials2d, summing):
  r_blk = 8192
  nrows = partials2d.shape[0]
  return pl.pallas_call(
      _bce_body,
      grid=(nrows // r_blk,),
      in_specs=[
          pl.BlockSpec((r_blk, 128), lambda i: (i, 0)),
          pl.BlockSpec((128, 8), lambda i: (0, 0)),
      ],
      out_specs=pl.BlockSpec(memory_space=pltpu.SMEM),
      out_shape=jax.ShapeDtypeStruct((1, 1), jnp.float32),
  )(partials2d, summing)


def _pad_to_rows(v, rows, fill):
  n = rows * 128 - v.shape[0]
  return jnp.concatenate(
      [v, jnp.full((n,), fill, v.dtype)]).reshape(rows, 128)


def kernel(x, edge_index, edges_pos, edges_neg, W):
  ei = edge_index.astype(jnp.int32)
  ep = edges_pos.astype(jnp.int32)
  en = edges_neg.astype(jnp.int32)

  h = _matmul(x, W)

  ei3 = jnp.stack([_pad_to_rows(ei[0], EI_ROWS, 0),
                   _pad_to_rows(ei[1], EI_ROWS, -1)])
  zinit = jnp.zeros((ZROWS_PER_TILE, D), jnp.float32)
  z = _segsum(h, ei3, zinit)

  a3 = jnp.concatenate([_pad_to_rows(ep[0], EI_ROWS, 0),
                        _pad_to_rows(en[0], EI_ROWS, 0)])
  b3 = jnp.concatenate([_pad_to_rows(ep[1], EI_ROWS, 0),
                        _pad_to_rows(en[1], EI_ROWS, 0)])
  eidx3 = jnp.stack([a3, b3])
  partials = _scores(z, eidx3).reshape(SC_ROWS * 128 * L // 128, 128)

  summing = (jnp.arange(128, dtype=jnp.int32)[:, None] // L ==
             jnp.arange(8, dtype=jnp.int32)[None, :]).astype(jnp.float32)
  return _bce(partials, summing)[0, 0]


# speedup vs baseline: 1.9210x; 1.4052x over previous
import jax
import jax.numpy as jnp
from jax import lax
from jax.experimental import pallas as pl
from jax.experimental.pallas import tpu as pltpu
from jax.experimental.pallas import tpu_sc as plsc

N_NODES = 10000
D = 128
N_EDGES = 320000

NC = 2
NS = 16
L = 16

N_HALF = N_NODES // NC
ZP = 5120
TRASH = 5100
ZROWS_PER_TILE = ZP // NS

EI_ROWS = 2560
K2_NCH = 80
K2_SUPER = EI_ROWS // 2

SC_ROWS = 5120
SC_ROWS_PER_W = SC_ROWS // (NC * NS)
K3_NCH = SC_ROWS_PER_W
E_HALF_PAD = SC_ROWS // 2 * 128


def _mesh():
  return plsc.VectorSubcoreMesh(
      core_axis_name="c", subcore_axis_name="s", num_cores=NC,
      num_subcores=NS)


def _mm_body(x_ref, w_ref, o_ref):
  o_ref[...] = lax.dot_general(
      x_ref[...], w_ref[...], (((1,), (0,)), ((), ())),
      preferred_element_type=jnp.float32,
      precision=lax.Precision.HIGHEST)


def _matmul(x, w):
  m_blk = 1000
  return pl.pallas_call(
      _mm_body,
      grid=(N_NODES // m_blk,),
      in_specs=[
          pl.BlockSpec((m_blk, D), lambda i: (i, 0)),
          pl.BlockSpec((D, D), lambda i: (0, 0)),
      ],
      out_specs=pl.BlockSpec((m_blk, D), lambda i: (i, 0)),
      out_shape=jax.ShapeDtypeStruct((N_NODES, D), jnp.float32),
  )(x, w)


def _segsum_body(h_ref, ei_ref, zinit_ref, z_out, sd0, sd1, loc0, loc1,
                 rows0, rows1, z_sh, sem_i, sem_g, sem_s):
  c = lax.axis_index("c")
  s = lax.axis_index("s")
  base = c * N_HALF
  sd = (sd0, sd1)
  loc = (loc0, loc1)
  rows = (rows0, rows1)

  pltpu.sync_copy(zinit_ref,
                  z_sh.at[pl.ds(s * ZROWS_PER_TILE, ZROWS_PER_TILE)])
  plsc.subcore_barrier()

  def idx_slice(k):
    r = jnp.minimum(s * K2_NCH + k, K2_SUPER - 1)
    return ei_ref.at[pl.ds(r * 4, 4)]

  def fire_idx(k, p):
    pltpu.async_copy(idx_slice(k), sd[p], sem_i)

  def wait_idx(p):
    pltpu.make_async_copy(idx_slice(0), sd[p], sem_i).wait()

  def fire_gather(p):
    for j in range(2):
      pltpu.async_copy(h_ref.at[sd[p].at[j]],
                       rows[p].at[pl.ds(j * 128, 128)], sem_g)

  def wait_gather(p):
    for j in range(2):
      pltpu.make_async_copy(h_ref.at[sd[p].at[j]],
                            rows[p].at[pl.ds(j * 128, 128)], sem_g).wait()

  def compute_loc(p):
    for j in range(2):
      for i in range(128 // L):
        d = sd[p][2 + j, pl.ds(i * L, L)]
        dl = d - base
        inb = (dl >= 0) & (dl < N_HALF)
        loc[p][j, pl.ds(i * L, L)] = jnp.where(inb, dl, TRASH)

  def fire_scatter(p):
    for j in range(2):
      pltpu.async_copy(rows[p].at[pl.ds(j * 128, 128)],
                       z_sh.at[loc[p].at[j]], sem_s, add=True)

  def wait_scatter(p):
    for j in range(2):
      pltpu.make_async_copy(rows[p].at[pl.ds(j * 128, 128)],
                            z_sh.at[loc[p].at[j]], sem_s).wait()

  fire_idx(0, 0)
  wait_idx(0)
  fire_gather(0)
  fire_idx(1, 1)
  wait_gather(0)
  wait_idx(1)
  fire_gather(1)
  compute_loc(0)
  fire_idx(2, 0)
  fire_scatter(0)

  def steady(k, p):
    q = 1 - p
    wait_gather(p)
    wait_idx(q)
    wait_scatter(q)
    fire_gather(q)
    compute_loc(p)
    fire_idx(k + 2, p)
    fire_scatter(p)

  def pair(m, carry):
    steady(2 * m + 1, 1)
    steady(2 * m + 2, 0)
    return carry

  lax.fori_loop(0, (K2_NCH - 2) // 2, pair, 0)

  wait_gather(1)
  wait_scatter(0)
  compute_loc(1)
  fire_scatter(1)
  wait_scatter(1)
  wait_idx(0)
  plsc.subcore_barrier()

  pltpu.sync_copy(
      z_sh.at[pl.ds(s * ZROWS_PER_TILE, ZROWS_PER_TILE)],
      z_out.at[pl.ds(c * ZP + s * ZROWS_PER_TILE, ZROWS_PER_TILE)])


def _segsum(h, ei4, zinit):
  return pl.kernel(
      _segsum_body,
      out_type=jax.ShapeDtypeStruct((NC * ZP, D), jnp.float32),
      mesh=_mesh(),
      scratch_types=[
          pltpu.VMEM((4, 128), jnp.int32),
          pltpu.VMEM((4, 128), jnp.int32),
          pltpu.VMEM((2, 128), jnp.int32),
          pltpu.VMEM((2, 128), jnp.int32),
          pltpu.VMEM((256, D), jnp.float32),
          pltpu.VMEM((256, D), jnp.float32),
          pltpu.VMEM_SHARED((ZP, D), jnp.float32),
          pltpu.SemaphoreType.DMA,
          pltpu.SemaphoreType.DMA,
          pltpu.SemaphoreType.DMA,
      ],
  )(h, ei4, zinit)


def _scores_body(z_ref, ed_ref, p_out, sd0, sd1, zab0, zab1, sv0, sv1,
                 sem_i, sem_g, sem_s):
  c = lax.axis_index("c")
  s = lax.axis_index("s")
  w = s * NC + c
  sd = (sd0, sd1)
  zab = (zab0, zab1)
  sv = (sv0, sv1)

  def idx_slice(k):
    r = jnp.minimum(w * K3_NCH + k, SC_ROWS - 1)
    return ed_ref.at[pl.ds(r * 2, 2)]

  def fire_idx(k, p):
    pltpu.async_copy(idx_slice(k), sd[p], sem_i)

  def wait_idx(p):
    pltpu.make_async_copy(idx_slice(0), sd[p], sem_i).wait()

  def adjust(p):
    for j in range(2):
      for i in range(128 // L):
        v = sd[p][j, pl.ds(i * L, L)]
        sd[p][j, pl.ds(i * L, L)] = jnp.where(
            v >= N_HALF, v + (ZP - N_HALF), v)

  def fire_gather(p):
    for j in range(2):
      pltpu.async_copy(z_ref.at[sd[p].at[j]],
                       zab[p].at[pl.ds(j * 128, 128)], sem_g)

  def wait_gather(p):
    for j in range(2):
      pltpu.make_async_copy(z_ref.at[sd[p].at[j]],
                            zab[p].at[pl.ds(j * 128, 128)], sem_g).wait()

  def compute(p):
    def group(g, carry):
      for t in range(L):
        e = g * L + t
        acc = zab[p][e, pl.ds(0, L)] * zab[p][128 + e, pl.ds(0, L)]
        for q in range(1, D // L):
          acc = acc + (zab[p][e, pl.ds(q * L, L)] *
                       zab[p][128 + e, pl.ds(q * L, L)])
        sv[p][pl.ds(e * L, L)] = acc
      return carry
    lax.fori_loop(0, 128 // L, group, 0)

  def fire_wb(k, p):
    pltpu.async_copy(sv[p], p_out.at[pl.ds((w * K3_NCH + k) * 2048, 2048)],
                     sem_s)

  def wait_wb(p):
    pltpu.make_async_copy(sv[p], p_out.at[pl.ds(0, 2048)], sem_s).wait()

  fire_idx(0, 0)
  wait_idx(0)
  adjust(0)
  fire_gather(0)
  fire_idx(1, 1)

  wait_gather(0)
  wait_idx(1)
  adjust(1)
  fire_gather(1)
  fire_idx(2, 0)
  compute(0)
  fire_wb(0, 0)

  wait_gather(1)
  wait_idx(0)
  adjust(0)
  fire_gather(0)
  fire_idx(3, 1)
  compute(1)
  fire_wb(1, 1)

  def steady(k, p):
    q = 1 - p
    wait_gather(p)
    wait_idx(q)
    adjust(q)
    fire_gather(q)
    fire_idx(k + 2, p)
    wait_wb(p)
    compute(p)
    fire_wb(k, p)

  def pair(m, carry):
    steady(2 * m + 2, 0)
    steady(2 * m + 3, 1)
    return carry

  lax.fori_loop(0, (K3_NCH - 4) // 2, pair, 0)

  wait_gather(0)
  wait_idx(1)
  adjust(1)
  fire_gather(1)
  fire_idx(K3_NCH, 0)
  wait_wb(0)
  compute(0)
  fire_wb(K3_NCH - 2, 0)

  wait_gather(1)
  wait_wb(1)
  compute(1)
  fire_wb(K3_NCH - 1, 1)

  wait_wb(0)
  wait_wb(1)
  wait_idx(0)


def _scores(z, ed):
  return pl.kernel(
      _scores_body,
      out_type=jax.ShapeDtypeStruct((SC_ROWS * 128 * L,), jnp.float32),
      mesh=_mesh(),
      scratch_types=[
          pltpu.VMEM((2, 128), jnp.int32),
          pltpu.VMEM((2, 128), jnp.int32),
          pltpu.VMEM((256, D), jnp.float32),
          pltpu.VMEM((256, D), jnp.float32),
          pltpu.VMEM((128 * L,), jnp.float32),
          pltpu.VMEM((128 * L,), jnp.float32),
          pltpu.SemaphoreType.DMA,
          pltpu.SemaphoreType.DMA,
          pltpu.SemaphoreType.DMA,
      ],
  )(z, ed)


def _bce_body(p_ref, m_ref, o_ref):
  i = pl.program_id(0)
  nrows = p_ref.shape[0]
  scores = lax.dot_general(
      p_ref[...], m_ref[...], (((1,), (0,)), ((), ())),
      preferred_element_type=jnp.float32,
      precision=lax.Precision.HIGHEST)
  r = lax.broadcasted_iota(jnp.int32, scores.shape, 0) + i * nrows
  g = lax.broadcasted_iota(jnp.int32, scores.shape, 1)
  eg = r * 8 + g
  is_pos = eg < E_HALF_PAD
  valid = (eg < N_EDGES) | ((eg >= E_HALF_PAD) &
                            (eg < E_HALF_PAD + N_EDGES))
  t = jnp.where(is_pos, 1.0, 0.0)
  p = scores
  term = jnp.maximum(p, 0.0) - p * t + jnp.log1p(jnp.exp(-jnp.abs(p)))
  term = jnp.where(valid, term, 0.0)
  bsum = jnp.sum(term)
  prev = jnp.where(i == 0, 0.0, o_ref[0, 0])
  tot = prev + bsum
  o_ref[0, 0] = jnp.where(i == pl.num_programs(0) - 1,
                          tot / (2.0 * N_EDGES), tot)


def _bce(partials2d, summing):
  r_blk = 8192
  nrows = partials2d.shape[0]
  return pl.pallas_call(
      _bce_body,
      grid=(nrows // r_blk,),
      in_specs=[
          pl.BlockSpec((r_blk, 128), lambda i: (i, 0)),
          pl.BlockSpec((128, 8), lambda i: (0, 0)),
      ],
      out_specs=pl.BlockSpec(memory_space=pltpu.SMEM),
      out_shape=jax.ShapeDtypeStruct((1, 1), jnp.float32),
  )(partials2d, summing)


def _pad_to_rows(v, rows, fill):
  n = rows * 128 - v.shape[0]
  return jnp.concatenate(
      [v, jnp.full((n,), fill, v.dtype)]).reshape(rows, 128)


def kernel(x, edge_index, edges_pos, edges_neg, W):
  ei = edge_index.astype(jnp.int32)
  ep = edges_pos.astype(jnp.int32)
  en = edges_neg.astype(jnp.int32)

  h = _matmul(x, W)

  src2 = _pad_to_rows(ei[0], EI_ROWS, 0).reshape(K2_SUPER, 2, 128)
  dst2 = _pad_to_rows(ei[1], EI_ROWS, -1).reshape(K2_SUPER, 2, 128)
  ei4 = jnp.concatenate([src2, dst2], axis=1).reshape(4 * K2_SUPER, 128)
  zinit = jnp.zeros((ZROWS_PER_TILE, D), jnp.float32)
  z = _segsum(h, ei4, zinit)

  a2 = jnp.concatenate([_pad_to_rows(ep[0], EI_ROWS, 0),
                        _pad_to_rows(en[0], EI_ROWS, 0)])
  b2 = jnp.concatenate([_pad_to_rows(ep[1], EI_ROWS, 0),
                        _pad_to_rows(en[1], EI_ROWS, 0)])
  ed = jnp.stack([a2, b2], axis=1).reshape(2 * SC_ROWS, 128)
  partials = _scores(z, ed).reshape(SC_ROWS * 16, 128)

  summing = (jnp.arange(128, dtype=jnp.int32)[:, None] // L ==
             jnp.arange(8, dtype=jnp.int32)[None, :]).astype(jnp.float32)
  return _bce(partials, summing)[0, 0]


# speedup vs baseline: 6.0777x; 3.1638x over previous
import jax
import jax.numpy as jnp
from jax import lax
from jax.experimental import pallas as pl
from jax.experimental.pallas import tpu as pltpu
from jax.experimental.pallas import tpu_sc as plsc

N_NODES = 10000
D = 128
N_EDGES = 320000

NC = 2
NS = 16
L = 16

N_HALF = N_NODES // NC
ZP = 5120
TRASH = 5100
ZROWS_PER_TILE = ZP // NS

EI_ROWS = 2560
K2_NCH = 80
K2_SUPER = EI_ROWS // 2

SC_ROWS = 5120
SC_ROWS_PER_W = SC_ROWS // (NC * NS)
K3_NCH = SC_ROWS_PER_W
E_HALF_PAD = SC_ROWS // 2 * 128


def _mesh():
  return plsc.VectorSubcoreMesh(
      core_axis_name="c", subcore_axis_name="s", num_cores=NC,
      num_subcores=NS)


def _mm_body(x_ref, w_ref, o_ref):
  o_ref[...] = lax.dot_general(
      x_ref[...], w_ref[...], (((1,), (0,)), ((), ())),
      preferred_element_type=jnp.float32,
      precision=lax.Precision.HIGHEST)


def _matmul(x, w):
  m_blk = 1000
  return pl.pallas_call(
      _mm_body,
      grid=(N_NODES // m_blk,),
      in_specs=[
          pl.BlockSpec((m_blk, D), lambda i: (i, 0)),
          pl.BlockSpec((D, D), lambda i: (0, 0)),
      ],
      out_specs=pl.BlockSpec((m_blk, D), lambda i: (i, 0)),
      out_shape=jax.ShapeDtypeStruct((N_NODES, D), jnp.float32),
  )(x, w)


def _segsum_body(h_ref, ei_ref, zinit_ref, z_out, sd0, sd1, loc0, loc1,
                 rows0, rows1, z_sh, sem_i, sem_g, sem_s):
  c = lax.axis_index("c")
  s = lax.axis_index("s")
  base = c * N_HALF
  sd = (sd0, sd1)
  loc = (loc0, loc1)
  rows = (rows0, rows1)

  pltpu.sync_copy(zinit_ref,
                  z_sh.at[pl.ds(s * ZROWS_PER_TILE, ZROWS_PER_TILE)])
  plsc.subcore_barrier()

  def idx_slice(k):
    r = jnp.minimum(s * K2_NCH + k, K2_SUPER - 1)
    return ei_ref.at[pl.ds(r * 4, 4)]

  def fire_idx(k, p):
    pltpu.async_copy(idx_slice(k), sd[p], sem_i)

  def wait_idx(p):
    pltpu.make_async_copy(idx_slice(0), sd[p], sem_i).wait()

  def fire_gather(p):
    for j in range(2):
      pltpu.async_copy(h_ref.at[sd[p].at[j]],
                       rows[p].at[pl.ds(j * 128, 128)], sem_g)

  def wait_gather(p):
    for j in range(2):
      pltpu.make_async_copy(h_ref.at[sd[p].at[j]],
                            rows[p].at[pl.ds(j * 128, 128)], sem_g).wait()

  def compute_loc(p):
    for j in range(2):
      for i in range(128 // L):
        d = sd[p][2 + j, pl.ds(i * L, L)]
        dl = d - base
        inb = (dl >= 0) & (dl < N_HALF)
        loc[p][j, pl.ds(i * L, L)] = jnp.where(inb, dl, TRASH)

  def fire_scatter(p):
    for j in range(2):
      pltpu.async_copy(rows[p].at[pl.ds(j * 128, 128)],
                       z_sh.at[loc[p].at[j]], sem_s, add=True)

  def wait_scatter(p):
    for j in range(2):
      pltpu.make_async_copy(rows[p].at[pl.ds(j * 128, 128)],
                            z_sh.at[loc[p].at[j]], sem_s).wait()

  fire_idx(0, 0)
  wait_idx(0)
  fire_gather(0)
  fire_idx(1, 1)
  wait_gather(0)
  wait_idx(1)
  fire_gather(1)
  compute_loc(0)
  fire_idx(2, 0)
  fire_scatter(0)

  def steady(k, p):
    q = 1 - p
    wait_gather(p)
    wait_idx(q)
    wait_scatter(q)
    fire_gather(q)
    compute_loc(p)
    fire_idx(k + 2, p)
    fire_scatter(p)

  def pair(m, carry):
    steady(2 * m + 1, 1)
    steady(2 * m + 2, 0)
    return carry

  lax.fori_loop(0, (K2_NCH - 2) // 2, pair, 0)

  wait_gather(1)
  wait_scatter(0)
  compute_loc(1)
  fire_scatter(1)
  wait_scatter(1)
  wait_idx(0)
  plsc.subcore_barrier()

  pltpu.sync_copy(
      z_sh.at[pl.ds(s * ZROWS_PER_TILE, ZROWS_PER_TILE)],
      z_out.at[pl.ds(c * ZP + s * ZROWS_PER_TILE, ZROWS_PER_TILE)])


def _segsum(h, ei4, zinit):
  return pl.kernel(
      _segsum_body,
      out_type=jax.ShapeDtypeStruct((NC * ZP, D), jnp.float32),
      mesh=_mesh(),
      scratch_types=[
          pltpu.VMEM((4, 128), jnp.int32),
          pltpu.VMEM((4, 128), jnp.int32),
          pltpu.VMEM((2, 128), jnp.int32),
          pltpu.VMEM((2, 128), jnp.int32),
          pltpu.VMEM((256, D), jnp.float32),
          pltpu.VMEM((256, D), jnp.float32),
          pltpu.VMEM_SHARED((ZP, D), jnp.float32),
          pltpu.SemaphoreType.DMA,
          pltpu.SemaphoreType.DMA,
          pltpu.SemaphoreType.DMA,
      ],
  )(h, ei4, zinit)


def _scores_body(z_ref, ed_ref, p_out, sd0, sd1, zab0, zab1, sv0, sv1,
                 sem_i, sem_g, sem_s):
  c = lax.axis_index("c")
  s = lax.axis_index("s")
  w = s * NC + c
  sd = (sd0, sd1)
  zab = (zab0, zab1)
  sv = (sv0, sv1)

  def idx_slice(k):
    r = jnp.minimum(w * K3_NCH + k, SC_ROWS - 1)
    return ed_ref.at[pl.ds(r * 2, 2)]

  def fire_idx(k, p):
    pltpu.async_copy(idx_slice(k), sd[p], sem_i)

  def wait_idx(p):
    pltpu.make_async_copy(idx_slice(0), sd[p], sem_i).wait()

  def adjust(p):
    for j in range(2):
      for i in range(128 // L):
        v = sd[p][j, pl.ds(i * L, L)]
        sd[p][j, pl.ds(i * L, L)] = jnp.where(
            v >= N_HALF, v + (ZP - N_HALF), v)

  def fire_gather(p):
    for j in range(2):
      pltpu.async_copy(z_ref.at[sd[p].at[j]],
                       zab[p].at[pl.ds(j * 128, 128)], sem_g)

  def wait_gather(p):
    for j in range(2):
      pltpu.make_async_copy(z_ref.at[sd[p].at[j]],
                            zab[p].at[pl.ds(j * 128, 128)], sem_g).wait()

  def compute(p):
    def group(g, carry):
      for t in range(L):
        e = g * L + t
        acc = zab[p][e, pl.ds(0, L)] * zab[p][128 + e, pl.ds(0, L)]
        for q in range(1, D // L):
          acc = acc + (zab[p][e, pl.ds(q * L, L)] *
                       zab[p][128 + e, pl.ds(q * L, L)])
        sv[p][pl.ds(e * L, L)] = acc
      return carry
    lax.fori_loop(0, 128 // L, group, 0)

  def fire_wb(k, p):
    pltpu.async_copy(sv[p], p_out.at[pl.ds((w * K3_NCH + k) * 2048, 2048)],
                     sem_s)

  def wait_wb(p):
    pltpu.make_async_copy(sv[p], p_out.at[pl.ds(0, 2048)], sem_s).wait()

  fire_idx(0, 0)
  wait_idx(0)
  adjust(0)
  fire_gather(0)
  fire_idx(1, 1)

  wait_gather(0)
  wait_idx(1)
  adjust(1)
  fire_gather(1)
  fire_idx(2, 0)
  compute(0)
  fire_wb(0, 0)

  wait_gather(1)
  wait_idx(0)
  adjust(0)
  fire_gather(0)
  fire_idx(3, 1)
  compute(1)
  fire_wb(1, 1)

  def steady(k, p):
    q = 1 - p
    wait_gather(p)
    wait_idx(q)
    adjust(q)
    fire_gather(q)
    fire_idx(k + 2, p)
    wait_wb(p)
    compute(p)
    fire_wb(k, p)

  def pair(m, carry):
    steady(2 * m + 2, 0)
    steady(2 * m + 3, 1)
    return carry

  lax.fori_loop(0, (K3_NCH - 4) // 2, pair, 0)

  wait_gather(0)
  wait_idx(1)
  adjust(1)
  fire_gather(1)
  fire_idx(K3_NCH, 0)
  wait_wb(0)
  compute(0)
  fire_wb(K3_NCH - 2, 0)

  wait_gather(1)
  wait_wb(1)
  compute(1)
  fire_wb(K3_NCH - 1, 1)

  wait_wb(0)
  wait_wb(1)
  wait_idx(0)


def _scores(z, ed):
  return pl.kernel(
      _scores_body,
      out_type=jax.ShapeDtypeStruct((SC_ROWS * 128 * L,), jnp.float32),
      mesh=_mesh(),
      scratch_types=[
          pltpu.VMEM((2, 128), jnp.int32),
          pltpu.VMEM((2, 128), jnp.int32),
          pltpu.VMEM((256, D), jnp.float32),
          pltpu.VMEM((256, D), jnp.float32),
          pltpu.VMEM((128 * L,), jnp.float32),
          pltpu.VMEM((128 * L,), jnp.float32),
          pltpu.SemaphoreType.DMA,
          pltpu.SemaphoreType.DMA,
          pltpu.SemaphoreType.DMA,
      ],
  )(z, ed)


def _bce_body(p_ref, m_ref, o_ref):
  i = pl.program_id(0)
  nrows = p_ref.shape[0]
  scores = lax.dot_general(
      p_ref[...], m_ref[...], (((1,), (0,)), ((), ())),
      preferred_element_type=jnp.float32,
      precision=lax.Precision.HIGHEST)
  r = lax.broadcasted_iota(jnp.int32, scores.shape, 0) + i * nrows
  g = lax.broadcasted_iota(jnp.int32, scores.shape, 1)
  eg = r * 8 + g
  is_pos = eg < E_HALF_PAD
  valid = (eg < N_EDGES) | ((eg >= E_HALF_PAD) &
                            (eg < E_HALF_PAD + N_EDGES))
  t = jnp.where(is_pos, 1.0, 0.0)
  p = scores
  term = jnp.maximum(p, 0.0) - p * t + jnp.log1p(jnp.exp(-jnp.abs(p)))
  term = jnp.where(valid, term, 0.0)
  bsum = jnp.sum(term)
  prev = jnp.where(i == 0, 0.0, o_ref[0, 0])
  tot = prev + bsum
  o_ref[0, 0] = jnp.where(i == pl.num_programs(0) - 1,
                          tot / (2.0 * N_EDGES), tot)


def _bce(partials2d, summing):
  r_blk = 8192
  nrows = partials2d.shape[0]
  return pl.pallas_call(
      _bce_body,
      grid=(nrows // r_blk,),
      in_specs=[
          pl.BlockSpec((r_blk, 128), lambda i: (i, 0)),
          pl.BlockSpec((128, 8), lambda i: (0, 0)),
      ],
      out_specs=pl.BlockSpec(memory_space=pltpu.SMEM),
      out_shape=jax.ShapeDtypeStruct((1, 1), jnp.float32),
  )(partials2d, summing)


def _pad_to_rows(v, rows, fill=None):
  n = rows * 128 - v.shape[0]
  if fill is None:
    pad = jnp.arange(n, dtype=v.dtype) * 37 % N_NODES
  else:
    pad = jnp.full((n,), fill, v.dtype)
  return jnp.concatenate([v, pad]).reshape(rows, 128)


def kernel(x, edge_index, edges_pos, edges_neg, W):
  ei = edge_index.astype(jnp.int32)
  ep = edges_pos.astype(jnp.int32)
  en = edges_neg.astype(jnp.int32)

  h = _matmul(x, W)

  src2 = _pad_to_rows(ei[0], EI_ROWS).reshape(K2_SUPER, 2, 128)
  dst2 = _pad_to_rows(ei[1], EI_ROWS, -1).reshape(K2_SUPER, 2, 128)
  ei4 = jnp.concatenate([src2, dst2], axis=1).reshape(4 * K2_SUPER, 128)
  zinit = jnp.zeros((ZROWS_PER_TILE, D), jnp.float32)
  z = _segsum(h, ei4, zinit)

  a2 = jnp.concatenate([_pad_to_rows(ep[0], EI_ROWS),
                        _pad_to_rows(en[0], EI_ROWS)])
  b2 = jnp.concatenate([_pad_to_rows(ep[1], EI_ROWS),
                        _pad_to_rows(en[1], EI_ROWS)])
  ed = jnp.stack([a2, b2], axis=1).reshape(2 * SC_ROWS, 128)
  partials = _scores(z, ed).reshape(SC_ROWS * 16, 128)

  summing = (jnp.arange(128, dtype=jnp.int32)[:, None] // L ==
             jnp.arange(8, dtype=jnp.int32)[None, :]).astype(jnp.float32)
  return _bce(partials, summing)[0, 0]


# speedup vs baseline: 6.4394x; 1.0595x over previous
import jax
import jax.numpy as jnp
from jax import lax
from jax.experimental import pallas as pl
from jax.experimental.pallas import tpu as pltpu
from jax.experimental.pallas import tpu_sc as plsc

N_NODES = 10000
D = 128
N_EDGES = 320000

NC = 2
NS = 16
L = 16

N_HALF = N_NODES // NC
ZP = 5120
ZR = NC * ZP
TRASH = 5100
ZROWS_PER_TILE = ZP // NS

EI_ROWS = 2560
K2_SUPER = EI_ROWS // 2
K2_NCH = K2_SUPER // NS

SC_ROWS = 5120
K3_NCH = SC_ROWS // (NC * NS)
E_HALF_PAD = SC_ROWS // 2 * 128


def _mesh():
  return plsc.VectorSubcoreMesh(
      core_axis_name="c", subcore_axis_name="s", num_cores=NC,
      num_subcores=NS)


def _mm_body(x_ref, w_ref, o_ref):
  o_ref[...] = lax.dot_general(
      x_ref[...], w_ref[...], (((1,), (0,)), ((), ())),
      preferred_element_type=jnp.float32,
      precision=lax.Precision.HIGHEST)


def _matmul(x, w):
  m_blk = 1000
  return pl.pallas_call(
      _mm_body,
      grid=(N_NODES // m_blk,),
      in_specs=[
          pl.BlockSpec((m_blk, D), lambda i: (i, 0)),
          pl.BlockSpec((D, D), lambda i: (0, 0)),
      ],
      out_specs=pl.BlockSpec((m_blk, D), lambda i: (i, 0)),
      out_shape=jax.ShapeDtypeStruct((N_NODES, D), jnp.float32),
  )(x, w)


def _segsum_body(h_ref, ei_ref, zinit_ref, z_out, sd0, sd1, loc0, loc1,
                 rows0, rows1, z_sh, sem_i, sem_g, sem_s):
  c = lax.axis_index("c")
  s = lax.axis_index("s")
  base = c * N_HALF
  sd = (sd0, sd1)
  loc = (loc0, loc1)
  rows = (rows0, rows1)

  pltpu.sync_copy(zinit_ref,
                  z_sh.at[pl.ds(s * ZROWS_PER_TILE, ZROWS_PER_TILE)])
  plsc.subcore_barrier()

  def idx_slice(k):
    r = jnp.minimum(s * K2_NCH + k, K2_SUPER - 1)
    return ei_ref.at[pl.ds(r * 4, 4)]

  def fire_idx(k, p):
    pltpu.async_copy(idx_slice(k), sd[p], sem_i)

  def wait_idx(p):
    pltpu.make_async_copy(idx_slice(0), sd[p], sem_i).wait()

  def fire_gather(p):
    for j in range(2):
      pltpu.async_copy(h_ref.at[sd[p].at[j]],
                       rows[p].at[pl.ds(j * 128, 128)], sem_g)

  def wait_gather(p):
    for j in range(2):
      pltpu.make_async_copy(h_ref.at[sd[p].at[j]],
                            rows[p].at[pl.ds(j * 128, 128)], sem_g).wait()

  def compute_loc(p):
    for j in range(2):
      for i in range(128 // L):
        d = sd[p][2 + j, pl.ds(i * L, L)]
        dl = d - base
        inb = (dl >= 0) & (dl < N_HALF)
        loc[p][j, pl.ds(i * L, L)] = jnp.where(inb, dl, TRASH)

  def fire_scatter(p):
    for j in range(2):
      pltpu.async_copy(rows[p].at[pl.ds(j * 128, 128)],
                       z_sh.at[loc[p].at[j]], sem_s, add=True)

  def wait_scatter(p):
    for j in range(2):
      pltpu.make_async_copy(rows[p].at[pl.ds(j * 128, 128)],
                            z_sh.at[loc[p].at[j]], sem_s).wait()

  fire_idx(0, 0)
  wait_idx(0)
  fire_gather(0)
  fire_idx(1, 1)
  wait_gather(0)
  wait_idx(1)
  fire_gather(1)
  compute_loc(0)
  fire_idx(2, 0)
  fire_scatter(0)

  def steady(k, p):
    q = 1 - p
    wait_gather(p)
    wait_idx(q)
    wait_scatter(q)
    fire_gather(q)
    compute_loc(p)
    fire_idx(k + 2, p)
    fire_scatter(p)

  def pair(m, carry):
    steady(2 * m + 1, 1)
    steady(2 * m + 2, 0)
    return carry

  lax.fori_loop(0, (K2_NCH - 2) // 2, pair, 0)

  wait_gather(1)
  wait_scatter(0)
  compute_loc(1)
  fire_scatter(1)
  wait_scatter(1)
  wait_idx(0)
  plsc.subcore_barrier()

  pltpu.sync_copy(
      z_sh.at[pl.ds(s * ZROWS_PER_TILE, ZROWS_PER_TILE)],
      z_out.at[pl.ds(c * ZP + s * ZROWS_PER_TILE, ZROWS_PER_TILE)])


def _segsum(h, ei4, zinit):
  return pl.kernel(
      _segsum_body,
      out_type=jax.ShapeDtypeStruct((ZR, D), jnp.float32),
      mesh=_mesh(),
      scratch_types=[
          pltpu.VMEM((4, 128), jnp.int32),
          pltpu.VMEM((4, 128), jnp.int32),
          pltpu.VMEM((2, 128), jnp.int32),
          pltpu.VMEM((2, 128), jnp.int32),
          pltpu.VMEM((256, D), jnp.float32),
          pltpu.VMEM((256, D), jnp.float32),
          pltpu.VMEM_SHARED((ZP, D), jnp.float32),
          pltpu.SemaphoreType.DMA,
          pltpu.SemaphoreType.DMA,
          pltpu.SemaphoreType.DMA,
      ],
  )(h, ei4, zinit)


_HI_MASK = -65536


def _scores_body(z_ref, ed_ref, p_out, sd0, sd1, zab0, zab1, sv0, sv1,
                 sem_i, sem_g, sem_s):
  c = lax.axis_index("c")
  s = lax.axis_index("s")
  w = s * NC + c
  sd = (sd0, sd1)
  zab = (zab0, zab1)
  sv = (sv0, sv1)

  def idx_slice(k):
    r = jnp.minimum(w * K3_NCH + k, SC_ROWS - 1)
    return ed_ref.at[pl.ds(r * 2, 2)]

  def fire_idx(k, p):
    pltpu.async_copy(idx_slice(k), sd[p], sem_i)

  def wait_idx(p):
    pltpu.make_async_copy(idx_slice(0), sd[p], sem_i).wait()

  def adjust(p):
    for j in range(2):
      for i in range(128 // L):
        v = sd[p][j, pl.ds(i * L, L)]
        sd[p][j, pl.ds(i * L, L)] = jnp.where(
            v >= N_HALF, v + (ZP - N_HALF), v)

  def fire_gather(p):
    for j in range(2):
      pltpu.async_copy(z_ref.at[sd[p].at[j]],
                       zab[p].at[pl.ds(j * 128, 128)], sem_g)

  def wait_gather(p):
    for j in range(2):
      pltpu.make_async_copy(z_ref.at[sd[p].at[j]],
                            zab[p].at[pl.ds(j * 128, 128)], sem_g).wait()

  def compute(p):
    def group(g, carry):
      for t in range(L):
        e = g * L + t
        acc = None
        for q in range(4):
          va = zab[p][e, pl.ds(q * L, L)]
          vb = zab[p][128 + e, pl.ds(q * L, L)]
          alo = lax.bitcast_convert_type(va << 16, jnp.float32)
          blo = lax.bitcast_convert_type(vb << 16, jnp.float32)
          ahi = lax.bitcast_convert_type(va & _HI_MASK, jnp.float32)
          bhi = lax.bitcast_convert_type(vb & _HI_MASK, jnp.float32)
          term = alo * blo + ahi * bhi
          acc = term if acc is None else acc + term
        sv[p][pl.ds(e * L, L)] = acc
      return carry
    lax.fori_loop(0, 128 // L, group, 0)

  def fire_wb(k, p):
    pltpu.async_copy(sv[p], p_out.at[pl.ds((w * K3_NCH + k) * 2048, 2048)],
                     sem_s)

  def wait_wb(p):
    pltpu.make_async_copy(sv[p], p_out.at[pl.ds(0, 2048)], sem_s).wait()

  fire_idx(0, 0)
  wait_idx(0)
  adjust(0)
  fire_gather(0)
  fire_idx(1, 1)

  wait_gather(0)
  wait_idx(1)
  adjust(1)
  fire_gather(1)
  fire_idx(2, 0)
  compute(0)
  fire_wb(0, 0)

  wait_gather(1)
  wait_idx(0)
  adjust(0)
  fire_gather(0)
  fire_idx(3, 1)
  compute(1)
  fire_wb(1, 1)

  def steady(k, p):
    q = 1 - p
    wait_gather(p)
    wait_idx(q)
    adjust(q)
    fire_gather(q)
    fire_idx(k + 2, p)
    wait_wb(p)
    compute(p)
    fire_wb(k, p)

  def pair(m, carry):
    steady(2 * m + 2, 0)
    steady(2 * m + 3, 1)
    return carry

  lax.fori_loop(0, (K3_NCH - 4) // 2, pair, 0)

  wait_gather(0)
  wait_idx(1)
  adjust(1)
  fire_gather(1)
  fire_idx(K3_NCH, 0)
  wait_wb(0)
  compute(0)
  fire_wb(K3_NCH - 2, 0)

  wait_gather(1)
  wait_wb(1)
  compute(1)
  fire_wb(K3_NCH - 1, 1)

  wait_wb(0)
  wait_wb(1)
  wait_idx(0)


def _scores(zp, ed):
  return pl.kernel(
      _scores_body,
      out_type=jax.ShapeDtypeStruct((SC_ROWS * 128 * L,), jnp.float32),
      mesh=_mesh(),
      compiler_params=pltpu.CompilerParams(use_tc_tiling_on_sc=False),
      scratch_types=[
          pltpu.VMEM((2, 128), jnp.int32),
          pltpu.VMEM((2, 128), jnp.int32),
          pltpu.VMEM((256, 64), jnp.int32),
          pltpu.VMEM((256, 64), jnp.int32),
          pltpu.VMEM((128 * L,), jnp.float32),
          pltpu.VMEM((128 * L,), jnp.float32),
          pltpu.SemaphoreType.DMA,
          pltpu.SemaphoreType.DMA,
          pltpu.SemaphoreType.DMA,
      ],
  )(zp, ed)


def _bce_body(p_ref, m_ref, o_ref):
  i = pl.program_id(0)
  nrows = p_ref.shape[0]
  scores = lax.dot_general(
      p_ref[...], m_ref[...], (((1,), (0,)), ((), ())),
      preferred_element_type=jnp.float32,
      precision=lax.Precision.HIGHEST)
  r = lax.broadcasted_iota(jnp.int32, scores.shape, 0) + i * nrows
  g = lax.broadcasted_iota(jnp.int32, scores.shape, 1)
  eg = r * 8 + g
  is_pos = eg < E_HALF_PAD
  valid = (eg < N_EDGES) | ((eg >= E_HALF_PAD) &
                            (eg < E_HALF_PAD + N_EDGES))
  t = jnp.where(is_pos, 1.0, 0.0)
  p = scores
  term = jnp.maximum(p, 0.0) - p * t + jnp.log1p(jnp.exp(-jnp.abs(p)))
  term = jnp.where(valid, term, 0.0)
  bsum = jnp.sum(term)
  prev = jnp.where(i == 0, 0.0, o_ref[0, 0])
  tot = prev + bsum
  o_ref[0, 0] = jnp.where(i == pl.num_programs(0) - 1,
                          tot / (2.0 * N_EDGES), tot)


def _bce(partials2d, summing):
  r_blk = 8192
  nrows = partials2d.shape[0]
  return pl.pallas_call(
      _bce_body,
      grid=(nrows // r_blk,),
      in_specs=[
          pl.BlockSpec((r_blk, 128), lambda i: (i, 0)),
          pl.BlockSpec((128, 8), lambda i: (0, 0)),
      ],
      out_specs=pl.BlockSpec(memory_space=pltpu.SMEM),
      out_shape=jax.ShapeDtypeStruct((1, 1), jnp.float32),
  )(partials2d, summing)


def _pad_to_rows(v, rows, base=0, mod=N_NODES):
  n = rows * 128 - v.shape[0]
  pad = base + jnp.arange(n, dtype=v.dtype) * 37 % mod
  return jnp.concatenate([v, pad]).reshape(rows, 128)


def kernel(x, edge_index, edges_pos, edges_neg, W):
  ei = edge_index.astype(jnp.int32)
  ep = edges_pos.astype(jnp.int32)
  en = edges_neg.astype(jnp.int32)

  h = _matmul(x, W)

  src2 = _pad_to_rows(ei[0], EI_ROWS).reshape(K2_SUPER, 2, 128)
  dst2 = _pad_to_rows(ei[1], EI_ROWS, base=-EI_ROWS * 128,
                      mod=N_NODES).reshape(K2_SUPER, 2, 128)
  ei4 = jnp.concatenate([src2, dst2], axis=1).reshape(4 * K2_SUPER, 128)
  zinit = jnp.zeros((ZROWS_PER_TILE, D), jnp.float32)
  z = _segsum(h, ei4, zinit)

  zp = lax.bitcast_convert_type(
      z.astype(jnp.bfloat16).reshape(ZR, 64, 2), jnp.int32)

  a2 = jnp.concatenate([_pad_to_rows(ep[0], EI_ROWS),
                        _pad_to_rows(en[0], EI_ROWS)])
  b2 = jnp.concatenate([_pad_to_rows(ep[1], EI_ROWS),
                        _pad_to_rows(en[1], EI_ROWS)])
  ed = jnp.stack([a2, b2], axis=1).reshape(2 * SC_ROWS, 128)
  partials = _scores(zp, ed).reshape(SC_ROWS * 16, 128)

  summing = (jnp.arange(128, dtype=jnp.int32)[:, None] // L ==
             jnp.arange(8, dtype=jnp.int32)[None, :]).astype(jnp.float32)
  return _bce(partials, summing)[0, 0]


# speedup vs baseline: 7.0453x; 1.0941x over previous
import jax
import jax.numpy as jnp
from jax import lax
from jax.experimental import pallas as pl
from jax.experimental.pallas import tpu as pltpu
from jax.experimental.pallas import tpu_sc as plsc

N_NODES = 10000
D = 128
N_EDGES = 320000

NC = 2
NS = 16
L = 16

N_HALF = N_NODES // NC
ZP = 5120
ZR = NC * ZP
TRASH = 5100
ZROWS_PER_TILE = ZP // NS

EI_ROWS = 2560
K2_SUPER = EI_ROWS // 2
K2_NCH = K2_SUPER // NS

SC_ROWS = 5120
K3_NCH = SC_ROWS // (NC * NS)
E_HALF_PAD = SC_ROWS // 2 * 128


def _mesh():
  return plsc.VectorSubcoreMesh(
      core_axis_name="c", subcore_axis_name="s", num_cores=NC,
      num_subcores=NS)


def _mm_body(x_ref, w_ref, o_ref):
  o_ref[...] = lax.dot_general(
      x_ref[...], w_ref[...], (((1,), (0,)), ((), ())),
      preferred_element_type=jnp.float32,
      precision=lax.Precision.HIGHEST).astype(jnp.bfloat16)


def _matmul(x, w):
  m_blk = 1000
  return pl.pallas_call(
      _mm_body,
      grid=(N_NODES // m_blk,),
      in_specs=[
          pl.BlockSpec((m_blk, D), lambda i: (i, 0)),
          pl.BlockSpec((D, D), lambda i: (0, 0)),
      ],
      out_specs=pl.BlockSpec((m_blk, D), lambda i: (i, 0)),
      out_shape=jax.ShapeDtypeStruct((N_NODES, D), jnp.bfloat16),
  )(x, w)


def _segsum_body(h_ref, ei_ref, zinit_ref, z_out, sd0, sd1, loc0, loc1,
                 rows0, rows1, z_sh, sem_i, sem_g, sem_s):
  c = lax.axis_index("c")
  s = lax.axis_index("s")
  base = c * N_HALF
  sd = (sd0, sd1)
  loc = (loc0, loc1)
  rows = (rows0, rows1)

  pltpu.sync_copy(zinit_ref,
                  z_sh.at[pl.ds(s * ZROWS_PER_TILE, ZROWS_PER_TILE)])
  plsc.subcore_barrier()

  def idx_slice(k):
    r = jnp.minimum(s * K2_NCH + k, K2_SUPER - 1)
    return ei_ref.at[pl.ds(r * 4, 4)]

  def fire_idx(k, p):
    pltpu.async_copy(idx_slice(k), sd[p], sem_i)

  def wait_idx(p):
    pltpu.make_async_copy(idx_slice(0), sd[p], sem_i).wait()

  def fire_gather(p):
    for j in range(2):
      pltpu.async_copy(h_ref.at[sd[p].at[j]],
                       rows[p].at[pl.ds(j * 128, 128)], sem_g)

  def wait_gather(p):
    for j in range(2):
      pltpu.make_async_copy(h_ref.at[sd[p].at[j]],
                            rows[p].at[pl.ds(j * 128, 128)], sem_g).wait()

  def compute_loc(p):
    for j in range(2):
      for i in range(128 // L):
        d = sd[p][2 + j, pl.ds(i * L, L)]
        dl = d - base
        inb = (dl >= 0) & (dl < N_HALF)
        loc[p][j, pl.ds(i * L, L)] = jnp.where(inb, dl, TRASH)

  def fire_scatter(p):
    for j in range(2):
      pltpu.async_copy(rows[p].at[pl.ds(j * 128, 128)],
                       z_sh.at[loc[p].at[j]], sem_s, add=True)

  def wait_scatter(p):
    for j in range(2):
      pltpu.make_async_copy(rows[p].at[pl.ds(j * 128, 128)],
                            z_sh.at[loc[p].at[j]], sem_s).wait()

  fire_idx(0, 0)
  wait_idx(0)
  fire_gather(0)
  fire_idx(1, 1)
  wait_gather(0)
  wait_idx(1)
  fire_gather(1)
  compute_loc(0)
  fire_idx(2, 0)
  fire_scatter(0)

  def steady(k, p):
    q = 1 - p
    wait_gather(p)
    wait_idx(q)
    wait_scatter(q)
    fire_gather(q)
    compute_loc(p)
    fire_idx(k + 2, p)
    fire_scatter(p)

  def pair(m, carry):
    steady(2 * m + 1, 1)
    steady(2 * m + 2, 0)
    return carry

  lax.fori_loop(0, (K2_NCH - 2) // 2, pair, 0)

  wait_gather(1)
  wait_scatter(0)
  compute_loc(1)
  fire_scatter(1)
  wait_scatter(1)
  wait_idx(0)
  plsc.subcore_barrier()

  pltpu.sync_copy(
      z_sh.at[pl.ds(s * ZROWS_PER_TILE, ZROWS_PER_TILE)],
      z_out.at[pl.ds(c * ZP + s * ZROWS_PER_TILE, ZROWS_PER_TILE)])


def _segsum(h, ei4, zinit):
  return pl.kernel(
      _segsum_body,
      out_type=jax.ShapeDtypeStruct((ZR, D), jnp.bfloat16),
      mesh=_mesh(),
      compiler_params=pltpu.CompilerParams(use_tc_tiling_on_sc=False),
      scratch_types=[
          pltpu.VMEM((4, 128), jnp.int32),
          pltpu.VMEM((4, 128), jnp.int32),
          pltpu.VMEM((2, 128), jnp.int32),
          pltpu.VMEM((2, 128), jnp.int32),
          pltpu.VMEM((256, D), jnp.bfloat16),
          pltpu.VMEM((256, D), jnp.bfloat16),
          pltpu.VMEM_SHARED((ZP, D), jnp.bfloat16),
          pltpu.SemaphoreType.DMA,
          pltpu.SemaphoreType.DMA,
          pltpu.SemaphoreType.DMA,
      ],
  )(h, ei4, zinit)


def _scores_body(z_ref, ed_ref, p_out, sd0, sd1, zab0, zab1, sv0, sv1,
                 sem_i, sem_g, sem_s):
  c = lax.axis_index("c")
  s = lax.axis_index("s")
  w = s * NC + c
  sd = (sd0, sd1)
  zab = (zab0, zab1)
  sv = (sv0, sv1)

  def idx_slice(k):
    r = jnp.minimum(w * K3_NCH + k, SC_ROWS - 1)
    return ed_ref.at[pl.ds(r * 2, 2)]

  def fire_idx(k, p):
    pltpu.async_copy(idx_slice(k), sd[p], sem_i)

  def wait_idx(p):
    pltpu.make_async_copy(idx_slice(0), sd[p], sem_i).wait()

  def adjust(p):
    for j in range(2):
      for i in range(128 // L):
        v = sd[p][j, pl.ds(i * L, L)]
        sd[p][j, pl.ds(i * L, L)] = jnp.where(
            v >= N_HALF, v + (ZP - N_HALF), v)

  def fire_gather(p):
    for j in range(2):
      pltpu.async_copy(z_ref.at[sd[p].at[j]],
                       zab[p].at[pl.ds(j * 128, 128)], sem_g)

  def wait_gather(p):
    for j in range(2):
      pltpu.make_async_copy(z_ref.at[sd[p].at[j]],
                            zab[p].at[pl.ds(j * 128, 128)], sem_g).wait()

  def compute(p):
    def group(g, carry):
      for t in range(L):
        e = g * L + t
        acc = None
        for q in range(4):
          va = zab[p][e, pl.ds(q * L, L)]
          vb = zab[p][128 + e, pl.ds(q * L, L)]
          alo = lax.bitcast_convert_type(va << 16, jnp.float32)
          blo = lax.bitcast_convert_type(vb << 16, jnp.float32)
          ahi = lax.bitcast_convert_type(va, jnp.float32)
          bhi = lax.bitcast_convert_type(vb, jnp.float32)
          term = alo * blo + ahi * bhi
          acc = term if acc is None else acc + term
        sv[p][pl.ds(e * L, L)] = acc
      return carry
    lax.fori_loop(0, 128 // L, group, 0)

  def fire_wb(k, p):
    pltpu.async_copy(sv[p], p_out.at[pl.ds((w * K3_NCH + k) * 2048, 2048)],
                     sem_s)

  def wait_wb(p):
    pltpu.make_async_copy(sv[p], p_out.at[pl.ds(0, 2048)], sem_s).wait()

  fire_idx(0, 0)
  wait_idx(0)
  adjust(0)
  fire_gather(0)
  fire_idx(1, 1)

  wait_gather(0)
  wait_idx(1)
  adjust(1)
  fire_gather(1)
  fire_idx(2, 0)
  compute(0)
  fire_wb(0, 0)

  wait_gather(1)
  wait_idx(0)
  adjust(0)
  fire_gather(0)
  fire_idx(3, 1)
  compute(1)
  fire_wb(1, 1)

  def steady(k, p):
    q = 1 - p
    wait_gather(p)
    wait_idx(q)
    adjust(q)
    fire_gather(q)
    fire_idx(k + 2, p)
    wait_wb(p)
    compute(p)
    fire_wb(k, p)

  def pair(m, carry):
    steady(2 * m + 2, 0)
    steady(2 * m + 3, 1)
    return carry

  lax.fori_loop(0, (K3_NCH - 4) // 2, pair, 0)

  wait_gather(0)
  wait_idx(1)
  adjust(1)
  fire_gather(1)
  fire_idx(K3_NCH, 0)
  wait_wb(0)
  compute(0)
  fire_wb(K3_NCH - 2, 0)

  wait_gather(1)
  wait_wb(1)
  compute(1)
  fire_wb(K3_NCH - 1, 1)

  wait_wb(0)
  wait_wb(1)
  wait_idx(0)


def _scores(zp, ed):
  return pl.kernel(
      _scores_body,
      out_type=jax.ShapeDtypeStruct((SC_ROWS * 128 * L,), jnp.float32),
      mesh=_mesh(),
      compiler_params=pltpu.CompilerParams(use_tc_tiling_on_sc=False),
      scratch_types=[
          pltpu.VMEM((2, 128), jnp.int32),
          pltpu.VMEM((2, 128), jnp.int32),
          pltpu.VMEM((256, 64), jnp.int32),
          pltpu.VMEM((256, 64), jnp.int32),
          pltpu.VMEM((128 * L,), jnp.float32),
          pltpu.VMEM((128 * L,), jnp.float32),
          pltpu.SemaphoreType.DMA,
          pltpu.SemaphoreType.DMA,
          pltpu.SemaphoreType.DMA,
      ],
  )(zp, ed)


def _bce_body(p_ref, m_ref, o_ref):
  i = pl.program_id(0)
  nrows = p_ref.shape[0]
  scores = lax.dot_general(
      p_ref[...], m_ref[...], (((1,), (0,)), ((), ())),
      preferred_element_type=jnp.float32,
      precision=lax.Precision.HIGHEST)
  r = lax.broadcasted_iota(jnp.int32, scores.shape, 0) + i * nrows
  g = lax.broadcasted_iota(jnp.int32, scores.shape, 1)
  eg = r * 8 + g
  is_pos = eg < E_HALF_PAD
  valid = (eg < N_EDGES) | ((eg >= E_HALF_PAD) &
                            (eg < E_HALF_PAD + N_EDGES))
  t = jnp.where(is_pos, 1.0, 0.0)
  p = scores
  term = jnp.maximum(p, 0.0) - p * t + jnp.log1p(jnp.exp(-jnp.abs(p)))
  term = jnp.where(valid, term, 0.0)
  bsum = jnp.sum(term)
  prev = jnp.where(i == 0, 0.0, o_ref[0, 0])
  tot = prev + bsum
  o_ref[0, 0] = jnp.where(i == pl.num_programs(0) - 1,
                          tot / (2.0 * N_EDGES), tot)


def _bce(partials2d, summing):
  r_blk = 8192
  nrows = partials2d.shape[0]
  return pl.pallas_call(
      _bce_body,
      grid=(nrows // r_blk,),
      in_specs=[
          pl.BlockSpec((r_blk, 128), lambda i: (i, 0)),
          pl.BlockSpec((128, 8), lambda i: (0, 0)),
      ],
      out_specs=pl.BlockSpec(memory_space=pltpu.SMEM),
      out_shape=jax.ShapeDtypeStruct((1, 1), jnp.float32),
  )(partials2d, summing)


def _pad_to_rows(v, rows, base=0, mod=N_NODES):
  n = rows * 128 - v.shape[0]
  pad = base + jnp.arange(n, dtype=v.dtype) * 37 % mod
  return jnp.concatenate([v, pad]).reshape(rows, 128)


def kernel(x, edge_index, edges_pos, edges_neg, W):
  ei = edge_index.astype(jnp.int32)
  ep = edges_pos.astype(jnp.int32)
  en = edges_neg.astype(jnp.int32)

  h = _matmul(x, W)

  src2 = _pad_to_rows(ei[0], EI_ROWS).reshape(K2_SUPER, 2, 128)
  dst2 = _pad_to_rows(ei[1], EI_ROWS, base=-EI_ROWS * 128,
                      mod=N_NODES).reshape(K2_SUPER, 2, 128)
  ei4 = jnp.concatenate([src2, dst2], axis=1).reshape(4 * K2_SUPER, 128)
  zinit = jnp.zeros((ZROWS_PER_TILE, D), jnp.bfloat16)
  z = _segsum(h, ei4, zinit)

  zp = lax.bitcast_convert_type(z.reshape(ZR, 64, 2), jnp.int32)

  a2 = jnp.concatenate([_pad_to_rows(ep[0], EI_ROWS),
                        _pad_to_rows(en[0], EI_ROWS)])
  b2 = jnp.concatenate([_pad_to_rows(ep[1], EI_ROWS),
                        _pad_to_rows(en[1], EI_ROWS)])
  ed = jnp.stack([a2, b2], axis=1).reshape(2 * SC_ROWS, 128)
  partials = _scores(zp, ed).reshape(SC_ROWS * 16, 128)

  summing = (jnp.arange(128, dtype=jnp.int32)[:, None] // L ==
             jnp.arange(8, dtype=jnp.int32)[None, :]).astype(jnp.float32)
  return _bce(partials, summing)[0, 0]


# speedup vs baseline: 7.3571x; 1.0443x over previous
import jax
import jax.numpy as jnp
from jax import lax
from jax.experimental import pallas as pl
from jax.experimental.pallas import tpu as pltpu
from jax.experimental.pallas import tpu_sc as plsc

N_NODES = 10000
D = 128
N_EDGES = 320000

NC = 2
NS = 16
L = 16

N_HALF = N_NODES // NC
ZP = 5120
ZR = NC * ZP
TRASH = 5100
ZROWS_PER_TILE = ZP // NS

EI_ROWS = 2560
K2_SUPER = EI_ROWS // 2
K2_NCH = K2_SUPER // NS

SC_ROWS = 5120
K3_NCH = SC_ROWS // (NC * NS)
E_HALF_PAD = SC_ROWS // 2 * 128


def _mesh():
  return plsc.VectorSubcoreMesh(
      core_axis_name="c", subcore_axis_name="s", num_cores=NC,
      num_subcores=NS)


def _mm_body(x_ref, w_ref, o_ref):
  o_ref[...] = lax.dot_general(
      x_ref[...], w_ref[...], (((1,), (0,)), ((), ())),
      preferred_element_type=jnp.float32,
      precision=lax.Precision.HIGHEST).astype(jnp.bfloat16)


def _matmul(x, w):
  m_blk = 1000
  return pl.pallas_call(
      _mm_body,
      grid=(N_NODES // m_blk,),
      in_specs=[
          pl.BlockSpec((m_blk, D), lambda i: (i, 0)),
          pl.BlockSpec((D, D), lambda i: (0, 0)),
      ],
      out_specs=pl.BlockSpec((m_blk, D), lambda i: (i, 0)),
      out_shape=jax.ShapeDtypeStruct((N_NODES, D), jnp.bfloat16),
  )(x, w)


def _segsum_body(h_ref, ei_ref, zinit_ref, z_out, sd0, sd1, loc0, loc1,
                 rows0, rows1, z_sh, sem_i, sem_g, sem_s):
  c = lax.axis_index("c")
  s = lax.axis_index("s")
  base = c * N_HALF
  sd = (sd0, sd1)
  loc = (loc0, loc1)
  rows = (rows0, rows1)

  pltpu.sync_copy(zinit_ref,
                  z_sh.at[pl.ds(s * ZROWS_PER_TILE, ZROWS_PER_TILE)])
  plsc.subcore_barrier()

  def idx_slice(k):
    r = jnp.minimum(s * K2_NCH + k, K2_SUPER - 1)
    return ei_ref.at[pl.ds(r * 4, 4)]

  def fire_idx(k, p):
    pltpu.async_copy(idx_slice(k), sd[p], sem_i)

  def wait_idx(p):
    pltpu.make_async_copy(idx_slice(0), sd[p], sem_i).wait()

  def fire_gather(p):
    for j in range(2):
      pltpu.async_copy(h_ref.at[sd[p].at[j]],
                       rows[p].at[pl.ds(j * 128, 128)], sem_g)

  def wait_gather(p):
    for j in range(2):
      pltpu.make_async_copy(h_ref.at[sd[p].at[j]],
                            rows[p].at[pl.ds(j * 128, 128)], sem_g).wait()

  def compute_loc(p):
    for j in range(2):
      for i in range(128 // L):
        d = sd[p][2 + j, pl.ds(i * L, L)]
        dl = d - base
        inb = (dl >= 0) & (dl < N_HALF)
        loc[p][j, pl.ds(i * L, L)] = jnp.where(inb, dl, TRASH)

  def fire_scatter(p):
    for j in range(2):
      pltpu.async_copy(rows[p].at[pl.ds(j * 128, 128)],
                       z_sh.at[loc[p].at[j]], sem_s, add=True)

  def wait_scatter(p):
    for j in range(2):
      pltpu.make_async_copy(rows[p].at[pl.ds(j * 128, 128)],
                            z_sh.at[loc[p].at[j]], sem_s).wait()

  fire_idx(0, 0)
  wait_idx(0)
  fire_gather(0)
  fire_idx(1, 1)
  wait_gather(0)
  wait_idx(1)
  fire_gather(1)
  compute_loc(0)
  fire_idx(2, 0)
  fire_scatter(0)

  def steady(k, p):
    q = 1 - p
    wait_gather(p)
    wait_idx(q)
    wait_scatter(q)
    fire_gather(q)
    compute_loc(p)
    fire_idx(k + 2, p)
    fire_scatter(p)

  def pair(m, carry):
    steady(2 * m + 1, 1)
    steady(2 * m + 2, 0)
    return carry

  lax.fori_loop(0, (K2_NCH - 2) // 2, pair, 0)

  wait_gather(1)
  wait_scatter(0)
  compute_loc(1)
  fire_scatter(1)
  wait_scatter(1)
  wait_idx(0)
  plsc.subcore_barrier()

  pltpu.sync_copy(
      z_sh.at[pl.ds(s * ZROWS_PER_TILE, ZROWS_PER_TILE)],
      z_out.at[pl.ds(c * ZP + s * ZROWS_PER_TILE, ZROWS_PER_TILE)])


def _segsum(h, ei4, zinit):
  return pl.kernel(
      _segsum_body,
      out_type=jax.ShapeDtypeStruct((ZR, D), jnp.bfloat16),
      mesh=_mesh(),
      compiler_params=pltpu.CompilerParams(use_tc_tiling_on_sc=False),
      scratch_types=[
          pltpu.VMEM((4, 128), jnp.int32),
          pltpu.VMEM((4, 128), jnp.int32),
          pltpu.VMEM((2, 128), jnp.int32),
          pltpu.VMEM((2, 128), jnp.int32),
          pltpu.VMEM((256, D), jnp.bfloat16),
          pltpu.VMEM((256, D), jnp.bfloat16),
          pltpu.VMEM_SHARED((ZP, D), jnp.bfloat16),
          pltpu.SemaphoreType.DMA,
          pltpu.SemaphoreType.DMA,
          pltpu.SemaphoreType.DMA,
      ],
  )(h, ei4, zinit)


def _scores_body(z_ref, ed_ref, p_out, sd0, sd1, zab0, zab1, sv0, sv1,
                 sem_i, sem_g, sem_s):
  c = lax.axis_index("c")
  s = lax.axis_index("s")
  w = s * NC + c
  sd = (sd0, sd1)
  zab = (zab0, zab1)
  sv = (sv0, sv1)

  def idx_slice(k):
    r = jnp.minimum(w * K3_NCH + k, SC_ROWS - 1)
    return ed_ref.at[pl.ds(r * 2, 2)]

  def fire_idx(k, p):
    pltpu.async_copy(idx_slice(k), sd[p], sem_i)

  def wait_idx(p):
    pltpu.make_async_copy(idx_slice(0), sd[p], sem_i).wait()

  def adjust(p):
    for j in range(2):
      for i in range(128 // L):
        v = sd[p][j, pl.ds(i * L, L)]
        sd[p][j, pl.ds(i * L, L)] = jnp.where(
            v >= N_HALF, v + (ZP - N_HALF), v)

  def fire_gather(p):
    for j in range(2):
      pltpu.async_copy(z_ref.at[sd[p].at[j]],
                       zab[p].at[pl.ds(j * 128, 128)], sem_g)

  def wait_gather(p):
    for j in range(2):
      pltpu.make_async_copy(z_ref.at[sd[p].at[j]],
                            zab[p].at[pl.ds(j * 128, 128)], sem_g).wait()

  def compute(p):
    def group(g, carry):
      for t in range(L):
        e = g * L + t
        acc = None
        for q in range(4):
          va = zab[p][e, pl.ds(q * L, L)]
          vb = zab[p][128 + e, pl.ds(q * L, L)]
          alo = lax.bitcast_convert_type(va << 16, jnp.float32)
          blo = lax.bitcast_convert_type(vb << 16, jnp.float32)
          ahi = lax.bitcast_convert_type(va, jnp.float32)
          bhi = lax.bitcast_convert_type(vb, jnp.float32)
          term = alo * blo + ahi * bhi
          acc = term if acc is None else acc + term
        sv[p][pl.ds(e * L, L)] = acc
      return carry
    lax.fori_loop(0, 128 // L, group, 0)

  def fire_wb(k, p):
    pltpu.async_copy(sv[p], p_out.at[pl.ds((w * K3_NCH + k) * 2048, 2048)],
                     sem_s)

  def wait_wb(p):
    pltpu.make_async_copy(sv[p], p_out.at[pl.ds(0, 2048)], sem_s).wait()

  fire_idx(0, 0)
  wait_idx(0)
  adjust(0)
  fire_gather(0)
  fire_idx(1, 1)

  wait_gather(0)
  wait_idx(1)
  adjust(1)
  fire_gather(1)
  fire_idx(2, 0)
  compute(0)
  fire_wb(0, 0)

  wait_gather(1)
  wait_idx(0)
  adjust(0)
  fire_gather(0)
  fire_idx(3, 1)
  compute(1)
  fire_wb(1, 1)

  def steady(k, p):
    q = 1 - p
    wait_gather(p)
    wait_idx(q)
    adjust(q)
    fire_gather(q)
    fire_idx(k + 2, p)
    wait_wb(p)
    compute(p)
    fire_wb(k, p)

  def pair(m, carry):
    steady(2 * m + 2, 0)
    steady(2 * m + 3, 1)
    return carry

  lax.fori_loop(0, (K3_NCH - 4) // 2, pair, 0)

  wait_gather(0)
  wait_idx(1)
  adjust(1)
  fire_gather(1)
  fire_idx(K3_NCH, 0)
  wait_wb(0)
  compute(0)
  fire_wb(K3_NCH - 2, 0)

  wait_gather(1)
  wait_wb(1)
  compute(1)
  fire_wb(K3_NCH - 1, 1)

  wait_wb(0)
  wait_wb(1)
  wait_idx(0)


def _scores(zp, ed):
  return pl.kernel(
      _scores_body,
      out_type=jax.ShapeDtypeStruct((SC_ROWS * 128 * L,), jnp.float32),
      mesh=_mesh(),
      compiler_params=pltpu.CompilerParams(use_tc_tiling_on_sc=False),
      scratch_types=[
          pltpu.VMEM((2, 128), jnp.int32),
          pltpu.VMEM((2, 128), jnp.int32),
          pltpu.VMEM((256, 64), jnp.int32),
          pltpu.VMEM((256, 64), jnp.int32),
          pltpu.VMEM((128 * L,), jnp.float32),
          pltpu.VMEM((128 * L,), jnp.float32),
          pltpu.SemaphoreType.DMA,
          pltpu.SemaphoreType.DMA,
          pltpu.SemaphoreType.DMA,
      ],
  )(zp, ed)


def _bce_body(p_ref, m_ref, o_ref):
  i = pl.program_id(0)
  nrows = p_ref.shape[0]
  scores = lax.dot_general(
      m_ref[...], p_ref[...], (((0,), (1,)), ((), ())),
      preferred_element_type=jnp.float32,
      precision=lax.Precision.HIGHEST)
  r = lax.broadcasted_iota(jnp.int32, scores.shape, 1) + i * nrows
  g = lax.broadcasted_iota(jnp.int32, scores.shape, 0)
  eg = r * 8 + g
  is_pos = eg < E_HALF_PAD
  valid = (eg < N_EDGES) | ((eg >= E_HALF_PAD) &
                            (eg < E_HALF_PAD + N_EDGES))
  t = jnp.where(is_pos, 1.0, 0.0)
  p = scores
  term = jnp.maximum(p, 0.0) - p * t + jnp.log1p(jnp.exp(-jnp.abs(p)))
  term = jnp.where(valid, term, 0.0)
  bsum = jnp.sum(term)
  prev = jnp.where(i == 0, 0.0, o_ref[0, 0])
  tot = prev + bsum
  o_ref[0, 0] = jnp.where(i == pl.num_programs(0) - 1,
                          tot / (2.0 * N_EDGES), tot)


def _bce(partials2d, summing):
  r_blk = 8192
  nrows = partials2d.shape[0]
  return pl.pallas_call(
      _bce_body,
      grid=(nrows // r_blk,),
      in_specs=[
          pl.BlockSpec((r_blk, 128), lambda i: (i, 0)),
          pl.BlockSpec((128, 8), lambda i: (0, 0)),
      ],
      out_specs=pl.BlockSpec(memory_space=pltpu.SMEM),
      out_shape=jax.ShapeDtypeStruct((1, 1), jnp.float32),
  )(partials2d, summing)


def _pad_to_rows(v, rows, base=0, mod=N_NODES):
  n = rows * 128 - v.shape[0]
  pad = base + jnp.arange(n, dtype=v.dtype) * 37 % mod
  return jnp.concatenate([v, pad]).reshape(rows, 128)


def kernel(x, edge_index, edges_pos, edges_neg, W):
  ei = edge_index.astype(jnp.int32)
  ep = edges_pos.astype(jnp.int32)
  en = edges_neg.astype(jnp.int32)

  h = _matmul(x, W)

  src2 = _pad_to_rows(ei[0], EI_ROWS).reshape(K2_SUPER, 2, 128)
  dst2 = _pad_to_rows(ei[1], EI_ROWS, base=-EI_ROWS * 128,
                      mod=N_NODES).reshape(K2_SUPER, 2, 128)
  ei4 = jnp.concatenate([src2, dst2], axis=1).reshape(4 * K2_SUPER, 128)
  zinit = jnp.zeros((ZROWS_PER_TILE, D), jnp.bfloat16)
  z = _segsum(h, ei4, zinit)

  zp = lax.bitcast_convert_type(z.reshape(ZR, 64, 2), jnp.int32)

  a2 = jnp.concatenate([_pad_to_rows(ep[0], EI_ROWS),
                        _pad_to_rows(en[0], EI_ROWS)])
  b2 = jnp.concatenate([_pad_to_rows(ep[1], EI_ROWS),
                        _pad_to_rows(en[1], EI_ROWS)])
  ed = jnp.stack([a2, b2], axis=1).reshape(2 * SC_ROWS, 128)
  partials = _scores(zp, ed).reshape(SC_ROWS * 16, 128)

  summing = (jnp.arange(128, dtype=jnp.int32)[:, None] // L ==
             jnp.arange(8, dtype=jnp.int32)[None, :]).astype(jnp.float32)
  return _bce(partials, summing)[0, 0]


# speedup vs baseline: 7.3763x; 1.0026x over previous
import jax
import jax.numpy as jnp
from jax import lax
from jax.experimental import pallas as pl
from jax.experimental.pallas import tpu as pltpu
from jax.experimental.pallas import tpu_sc as plsc

N_NODES = 10000
D = 128
N_EDGES = 320000

NC = 2
NS = 16
L = 16

N_HALF = N_NODES // NC
ZP = 5120
ZR = NC * ZP
TRASH = 5100
ZROWS_PER_TILE = ZP // NS

EI_ROWS = 2560
K2_SUPER = EI_ROWS // 2
K2_NCH = K2_SUPER // NS

SC_ROWS = 5120
K3_NCH = SC_ROWS // (2 * NC * NS)
E_HALF_PAD = SC_ROWS // 2 * 128


def _mesh():
  return plsc.VectorSubcoreMesh(
      core_axis_name="c", subcore_axis_name="s", num_cores=NC,
      num_subcores=NS)


def _mm_body(x_ref, w_ref, o_ref):
  o_ref[...] = lax.dot_general(
      x_ref[...], w_ref[...], (((1,), (0,)), ((), ())),
      preferred_element_type=jnp.float32,
      precision=lax.Precision.HIGHEST).astype(jnp.bfloat16)


def _matmul(x, w):
  m_blk = 1000
  return pl.pallas_call(
      _mm_body,
      grid=(N_NODES // m_blk,),
      in_specs=[
          pl.BlockSpec((m_blk, D), lambda i: (i, 0)),
          pl.BlockSpec((D, D), lambda i: (0, 0)),
      ],
      out_specs=pl.BlockSpec((m_blk, D), lambda i: (i, 0)),
      out_shape=jax.ShapeDtypeStruct((N_NODES, D), jnp.bfloat16),
  )(x, w)


def _segsum_body(h_ref, ei_ref, zinit_ref, z_out, sd0, sd1, loc0, loc1,
                 rows0, rows1, z_sh, sem_i, sem_g, sem_s):
  c = lax.axis_index("c")
  s = lax.axis_index("s")
  base = c * N_HALF
  sd = (sd0, sd1)
  loc = (loc0, loc1)
  rows = (rows0, rows1)

  pltpu.sync_copy(zinit_ref,
                  z_sh.at[pl.ds(s * ZROWS_PER_TILE, ZROWS_PER_TILE)])
  plsc.subcore_barrier()

  def idx_slice(k):
    r = jnp.minimum(s * K2_NCH + k, K2_SUPER - 1)
    return ei_ref.at[pl.ds(r * 4, 4)]

  def fire_idx(k, p):
    pltpu.async_copy(idx_slice(k), sd[p], sem_i)

  def wait_idx(p):
    pltpu.make_async_copy(idx_slice(0), sd[p], sem_i).wait()

  def fire_gather(p):
    for j in range(2):
      pltpu.async_copy(h_ref.at[sd[p].at[j]],
                       rows[p].at[pl.ds(j * 128, 128)], sem_g)

  def wait_gather(p):
    for j in range(2):
      pltpu.make_async_copy(h_ref.at[sd[p].at[j]],
                            rows[p].at[pl.ds(j * 128, 128)], sem_g).wait()

  def compute_loc(p):
    for j in range(2):
      for i in range(128 // L):
        d = sd[p][2 + j, pl.ds(i * L, L)]
        dl = d - base
        inb = (dl >= 0) & (dl < N_HALF)
        loc[p][j, pl.ds(i * L, L)] = jnp.where(inb, dl, TRASH)

  def fire_scatter(p):
    for j in range(2):
      pltpu.async_copy(rows[p].at[pl.ds(j * 128, 128)],
                       z_sh.at[loc[p].at[j]], sem_s, add=True)

  def wait_scatter(p):
    for j in range(2):
      pltpu.make_async_copy(rows[p].at[pl.ds(j * 128, 128)],
                            z_sh.at[loc[p].at[j]], sem_s).wait()

  fire_idx(0, 0)
  wait_idx(0)
  fire_gather(0)
  fire_idx(1, 1)
  wait_gather(0)
  wait_idx(1)
  fire_gather(1)
  compute_loc(0)
  fire_idx(2, 0)
  fire_scatter(0)

  def steady(k, p):
    q = 1 - p
    wait_gather(p)
    wait_idx(q)
    wait_scatter(q)
    fire_gather(q)
    compute_loc(p)
    fire_idx(k + 2, p)
    fire_scatter(p)

  def pair(m, carry):
    steady(2 * m + 1, 1)
    steady(2 * m + 2, 0)
    return carry

  lax.fori_loop(0, (K2_NCH - 2) // 2, pair, 0)

  wait_gather(1)
  wait_scatter(0)
  compute_loc(1)
  fire_scatter(1)
  wait_scatter(1)
  wait_idx(0)
  plsc.subcore_barrier()

  pltpu.sync_copy(
      z_sh.at[pl.ds(s * ZROWS_PER_TILE, ZROWS_PER_TILE)],
      z_out.at[pl.ds(c * ZP + s * ZROWS_PER_TILE, ZROWS_PER_TILE)])


def _segsum(h, ei4, zinit):
  return pl.kernel(
      _segsum_body,
      out_type=jax.ShapeDtypeStruct((ZR, D), jnp.bfloat16),
      mesh=_mesh(),
      compiler_params=pltpu.CompilerParams(use_tc_tiling_on_sc=False),
      scratch_types=[
          pltpu.VMEM((4, 128), jnp.int32),
          pltpu.VMEM((4, 128), jnp.int32),
          pltpu.VMEM((2, 128), jnp.int32),
          pltpu.VMEM((2, 128), jnp.int32),
          pltpu.VMEM((256, D), jnp.bfloat16),
          pltpu.VMEM((256, D), jnp.bfloat16),
          pltpu.VMEM_SHARED((ZP, D), jnp.bfloat16),
          pltpu.SemaphoreType.DMA,
          pltpu.SemaphoreType.DMA,
          pltpu.SemaphoreType.DMA,
      ],
  )(h, ei4, zinit)


def _scores_body(z_ref, ed_ref, p_out, sd0, sd1, zab0, zab1, sv0, sv1,
                 sem_i, sem_g, sem_s):
  c = lax.axis_index("c")
  s = lax.axis_index("s")
  w = s * NC + c
  sd = (sd0, sd1)
  zab = (zab0, zab1)
  sv = (sv0, sv1)

  def idx_slice(k):
    r = jnp.minimum(w * K3_NCH + k, SC_ROWS // 2 - 1)
    return ed_ref.at[pl.ds(r * 4, 4)]

  def fire_idx(k, p):
    pltpu.async_copy(idx_slice(k), sd[p], sem_i)

  def wait_idx(p):
    pltpu.make_async_copy(idx_slice(0), sd[p], sem_i).wait()

  def adjust(p):
    for j in range(4):
      for i in range(128 // L):
        v = sd[p][j, pl.ds(i * L, L)]
        sd[p][j, pl.ds(i * L, L)] = jnp.where(
            v >= N_HALF, v + (ZP - N_HALF), v)

  def fire_gather(p):
    for j in range(4):
      pltpu.async_copy(z_ref.at[sd[p].at[j]],
                       zab[p].at[pl.ds(j * 128, 128)], sem_g)

  def wait_gather(p):
    for j in range(4):
      pltpu.make_async_copy(z_ref.at[sd[p].at[j]],
                            zab[p].at[pl.ds(j * 128, 128)], sem_g).wait()

  def compute(p):
    def group(g, carry):
      h = g >> 3
      ar = (h << 8) + ((g & 7) << 4)
      for t in range(L):
        e = ar + t
        acc = None
        for q in range(4):
          va = zab[p][e, pl.ds(q * L, L)]
          vb = zab[p][128 + e, pl.ds(q * L, L)]
          alo = lax.bitcast_convert_type(va << 16, jnp.float32)
          blo = lax.bitcast_convert_type(vb << 16, jnp.float32)
          ahi = lax.bitcast_convert_type(va, jnp.float32)
          bhi = lax.bitcast_convert_type(vb, jnp.float32)
          term = alo * blo + ahi * bhi
          acc = term if acc is None else acc + term
        sv[p][pl.ds(((g << 4) + t) * L, L)] = acc
      return carry
    lax.fori_loop(0, 256 // L, group, 0)

  def fire_wb(k, p):
    pltpu.async_copy(sv[p], p_out.at[pl.ds((w * K3_NCH + k) * 4096, 4096)],
                     sem_s)

  def wait_wb(p):
    pltpu.make_async_copy(sv[p], p_out.at[pl.ds(0, 4096)], sem_s).wait()

  fire_idx(0, 0)
  wait_idx(0)
  adjust(0)
  fire_gather(0)
  fire_idx(1, 1)

  wait_gather(0)
  wait_idx(1)
  adjust(1)
  fire_gather(1)
  fire_idx(2, 0)
  compute(0)
  fire_wb(0, 0)

  wait_gather(1)
  wait_idx(0)
  adjust(0)
  fire_gather(0)
  fire_idx(3, 1)
  compute(1)
  fire_wb(1, 1)

  def steady(k, p):
    q = 1 - p
    wait_gather(p)
    wait_idx(q)
    adjust(q)
    fire_gather(q)
    fire_idx(k + 2, p)
    wait_wb(p)
    compute(p)
    fire_wb(k, p)

  def pair(m, carry):
    steady(2 * m + 2, 0)
    steady(2 * m + 3, 1)
    return carry

  lax.fori_loop(0, (K3_NCH - 4) // 2, pair, 0)

  wait_gather(0)
  wait_idx(1)
  adjust(1)
  fire_gather(1)
  fire_idx(K3_NCH, 0)
  wait_wb(0)
  compute(0)
  fire_wb(K3_NCH - 2, 0)

  wait_gather(1)
  wait_wb(1)
  compute(1)
  fire_wb(K3_NCH - 1, 1)

  wait_wb(0)
  wait_wb(1)
  wait_idx(0)


def _scores(zp, ed):
  return pl.kernel(
      _scores_body,
      out_type=jax.ShapeDtypeStruct((SC_ROWS * 128 * L,), jnp.float32),
      mesh=_mesh(),
      compiler_params=pltpu.CompilerParams(use_tc_tiling_on_sc=False),
      scratch_types=[
          pltpu.VMEM((4, 128), jnp.int32),
          pltpu.VMEM((4, 128), jnp.int32),
          pltpu.VMEM((512, 64), jnp.int32),
          pltpu.VMEM((512, 64), jnp.int32),
          pltpu.VMEM((256 * L,), jnp.float32),
          pltpu.VMEM((256 * L,), jnp.float32),
          pltpu.SemaphoreType.DMA,
          pltpu.SemaphoreType.DMA,
          pltpu.SemaphoreType.DMA,
      ],
  )(zp, ed)


def _bce_body(p_ref, m_ref, o_ref):
  i = pl.program_id(0)
  nrows = p_ref.shape[0]
  scores = lax.dot_general(
      m_ref[...], p_ref[...], (((0,), (1,)), ((), ())),
      preferred_element_type=jnp.float32,
      precision=lax.Precision.HIGHEST)
  r = lax.broadcasted_iota(jnp.int32, scores.shape, 1) + i * nrows
  g = lax.broadcasted_iota(jnp.int32, scores.shape, 0)
  eg = r * 8 + g
  is_pos = eg < E_HALF_PAD
  valid = (eg < N_EDGES) | ((eg >= E_HALF_PAD) &
                            (eg < E_HALF_PAD + N_EDGES))
  t = jnp.where(is_pos, 1.0, 0.0)
  p = scores
  term = jnp.maximum(p, 0.0) - p * t + jnp.log1p(jnp.exp(-jnp.abs(p)))
  term = jnp.where(valid, term, 0.0)
  bsum = jnp.sum(term)
  prev = jnp.where(i == 0, 0.0, o_ref[0, 0])
  tot = prev + bsum
  o_ref[0, 0] = jnp.where(i == pl.num_programs(0) - 1,
                          tot / (2.0 * N_EDGES), tot)


def _bce(partials2d, summing):
  r_blk = 8192
  nrows = partials2d.shape[0]
  return pl.pallas_call(
      _bce_body,
      grid=(nrows // r_blk,),
      in_specs=[
          pl.BlockSpec((r_blk, 128), lambda i: (i, 0)),
          pl.BlockSpec((128, 8), lambda i: (0, 0)),
      ],
      out_specs=pl.BlockSpec(memory_space=pltpu.SMEM),
      out_shape=jax.ShapeDtypeStruct((1, 1), jnp.float32),
  )(partials2d, summing)


def _pad_to_rows(v, rows, base=0, mod=N_NODES):
  n = rows * 128 - v.shape[0]
  pad = base + jnp.arange(n, dtype=v.dtype) * 37 % mod
  return jnp.concatenate([v, pad]).reshape(rows, 128)


def kernel(x, edge_index, edges_pos, edges_neg, W):
  ei = edge_index.astype(jnp.int32)
  ep = edges_pos.astype(jnp.int32)
  en = edges_neg.astype(jnp.int32)

  h = _matmul(x, W)

  src2 = _pad_to_rows(ei[0], EI_ROWS).reshape(K2_SUPER, 2, 128)
  dst2 = _pad_to_rows(ei[1], EI_ROWS, base=-EI_ROWS * 128,
                      mod=N_NODES).reshape(K2_SUPER, 2, 128)
  ei4 = jnp.concatenate([src2, dst2], axis=1).reshape(4 * K2_SUPER, 128)
  zinit = jnp.zeros((ZROWS_PER_TILE, D), jnp.bfloat16)
  z = _segsum(h, ei4, zinit)

  zp = lax.bitcast_convert_type(z.reshape(ZR, 64, 2), jnp.int32)

  a2 = jnp.concatenate([_pad_to_rows(ep[0], EI_ROWS),
                        _pad_to_rows(en[0], EI_ROWS)])
  b2 = jnp.concatenate([_pad_to_rows(ep[1], EI_ROWS),
                        _pad_to_rows(en[1], EI_ROWS)])
  ed = jnp.stack([a2, b2], axis=1).reshape(2 * SC_ROWS, 128)
  partials = _scores(zp, ed).reshape(SC_ROWS * 16, 128)

  summing = (jnp.arange(128, dtype=jnp.int32)[:, None] // L ==
             jnp.arange(8, dtype=jnp.int32)[None, :]).astype(jnp.float32)
  return _bce(partials, summing)[0, 0]
